# baseline (speedup 1.0000x reference)
import functools

import jax
import jax.numpy as jnp
from jax import lax
from jax.experimental import pallas as pl
from jax.experimental.pallas import tpu as pltpu
from jax.experimental.pallas import tpu_sc as plsc

BATCH = 16384
EMB = 32
PACK = 128 // EMB

_info = plsc.get_sparse_core_info()
NC, NS, L = _info.num_cores, _info.num_subcores, _info.num_lanes
NW = NC * NS
B_PER_W = BATCH // NW
HALF = B_PER_W // 2
N_GROUP = HALF // L

_mesh = plsc.VectorSubcoreMesh(core_axis_name="c", subcore_axis_name="s")


@functools.partial(
    pl.kernel,
    mesh=_mesh,
    out_type=jax.ShapeDtypeStruct((BATCH,), jnp.float32),
    scratch_types=[
        pltpu.VMEM((B_PER_W,), jnp.int32),
        pltpu.VMEM((B_PER_W,), jnp.int32),
        pltpu.VMEM((B_PER_W,), jnp.int32),
        pltpu.VMEM((B_PER_W,), jnp.int32),
        pltpu.VMEM((HALF, 128), jnp.float32),
        pltpu.VMEM((HALF, 128), jnp.float32),
        pltpu.VMEM((B_PER_W,), jnp.float32),
        pltpu.SemaphoreType.DMA,
    ],
    compiler_params=pltpu.CompilerParams(needs_layout_passes=False),
)
def _mf_sc(ublk, uoff, vblk, voff, ue_hbm, ie_hbm, out_hbm,
           bu, ou, bv, ov, blku, blkv, out_v, sem):
    wid = lax.axis_index("s") * NC + lax.axis_index("c")
    base = wid * B_PER_W

    pltpu.sync_copy(ublk.at[pl.ds(base, B_PER_W)], bu)
    pltpu.sync_copy(uoff.at[pl.ds(base, B_PER_W)], ou)
    pltpu.sync_copy(vblk.at[pl.ds(base, B_PER_W)], bv)
    pltpu.sync_copy(voff.at[pl.ds(base, B_PER_W)], ov)

    iota = lax.broadcasted_iota(jnp.int32, (L,), 0)

    for h in range(2):
        copies = []
        for j in range(HALF // 128):
            s = h * HALF + j * 128
            d = j * 128
            copies.append(pltpu.async_copy(
                ue_hbm.at[bu.at[pl.ds(s, 128)]],
                blku.at[pl.ds(d, 128)], sem))
            copies.append(pltpu.async_copy(
                ie_hbm.at[bv.at[pl.ds(s, 128)]],
                blkv.at[pl.ds(d, 128)], sem))
        for cp in copies:
            cp.wait()

        def g_body(g, carry, h=h):
            rowv = g * L + iota
            ouv = ou[pl.ds(h * HALF + g * L, L)]
            ovv = ov[pl.ds(h * HALF + g * L, L)]
            acc = jnp.zeros((L,), jnp.float32)
            for c in range(EMB):
                gu = plsc.load_gather(blku, [rowv, ouv + c])
                gv = plsc.load_gather(blkv, [rowv, ovv + c])
                acc = acc + gu * gv
            out_v[pl.ds(h * HALF + g * L, L)] = acc
            return carry

        lax.fori_loop(0, N_GROUP, g_body, 0)

    pltpu.sync_copy(out_v, out_hbm.at[pl.ds(base, B_PER_W)])


def kernel(u, v, users_emb, items_emb):
    u = u.astype(jnp.int32)
    v = v.astype(jnp.int32)
    ublk = u // PACK
    uoff = (u % PACK) * EMB
    vblk = v // PACK
    voff = (v % PACK) * EMB
    ue4 = users_emb.reshape(-1, 128)
    ie4 = items_emb.reshape(-1, 128)
    return _mf_sc(ublk, uoff, vblk, voff, ue4, ie4)
